# baseline (speedup 1.0000x reference)
import jax
import jax.numpy as jnp
from jax import lax
from jax.experimental import pallas as pl
from jax.experimental.pallas import tpu as pltpu
from jax.experimental.pallas import tpu_sc as plsc

VOCAB = 1000000
EMBED_DIM = 32
BATCH = 16384
HIST = 50

NC, NS = 2, 16
NW = NC * NS
ROWS_W = BATCH // NW
CR = 8
N_CHUNKS = ROWS_W // CR
FLAT = CR * HIST
NBUF = 4
N_SUPER = N_CHUNKS // NBUF


def _rsqrt_newton(s):
    s = jnp.maximum(s, jnp.float32(1e-24))
    i = lax.bitcast_convert_type(s, jnp.int32)
    y = lax.bitcast_convert_type(jnp.int32(0x5F3759DF) - (i >> 1), jnp.float32)
    half_s = jnp.float32(0.5) * s
    for _ in range(2):
        y = y * (jnp.float32(1.5) - half_s * y * y)
    return y


def _sc_body(x_hbm, w_hbm, out_hbm, idxa, rows4, g0, g1, g2, g3, o0, o1, o2, o3):
    wid = lax.axis_index("s") * NC + lax.axis_index("c")
    gsem = (g0, g1, g2, g3)
    osem = (o0, o1, o2, o3)
    row0 = wid * ROWS_W

    pltpu.sync_copy(x_hbm.at[pl.ds(row0, ROWS_W), :], idxa)

    def fire_gather(c, b):
        for j in range(CR):
            pltpu.async_copy(
                w_hbm.at[idxa.at[c * CR + j]], rows4.at[b, j], gsem[b]
            )

    def wait_gather(b):
        for j in range(CR):
            pltpu.make_async_copy(
                w_hbm.at[pl.ds(0, HIST), :], rows4.at[b, j], gsem[b]
            ).wait()

    def out_copy(c, b):
        return pltpu.make_async_copy(
            rows4.at[b], out_hbm.at[pl.ds(row0 + c * CR, CR), :, :], osem[b]
        )

    def normalize(b):
        zeros = jnp.zeros((16,), jnp.int32)
        lane_off = lax.iota(jnp.int32, 16) * EMBED_DIM

        def blk_body(bk, carry):
            bvec = lane_off + bk * (16 * EMBED_DIM)
            acc = None
            for g in range(0, EMBED_DIM, 8):
                cols = [
                    plsc.load_gather(rows4.at[b], [zeros, zeros, bvec + d])
                    for d in range(g, g + 8)
                ]
                sq = [c * c for c in cols]
                while len(sq) > 1:
                    sq = [sq[k] + sq[k + 1] for k in range(0, len(sq), 2)]
                acc = sq[0] if acc is None else acc + sq[0]
            y = _rsqrt_newton(acc)
            for g in range(0, EMBED_DIM, 8):
                for d in range(g, g + 8):
                    v = plsc.load_gather(rows4.at[b], [zeros, zeros, bvec + d])
                    plsc.store_scatter(
                        rows4.at[b], [zeros, zeros, bvec + d], v * y
                    )
            return carry

        lax.fori_loop(0, FLAT // 16, blk_body, 0)

    fire_gather(0, 0)
    fire_gather(1, 1)

    def super_body(s, carry):
        for i in range(NBUF):
            c = s * NBUF + i
            wait_gather(i)
            normalize(i)
            out_copy(c, i).start()
            bn = (i + 2) % NBUF

            @pl.when(c + 2 < N_CHUNKS)
            def _():
                @pl.when(c >= 2)
                def _():
                    out_copy(c - 2, bn).wait()

                fire_gather(c + 2, bn)

        return carry

    lax.fori_loop(0, N_SUPER, super_body, 0)
    for c in range(N_CHUNKS - NBUF, N_CHUNKS):
        out_copy(c, c % NBUF).wait()


@jax.jit
def kernel(x, weight):
    out = pl.kernel(
        _sc_body,
        out_type=jax.ShapeDtypeStruct((BATCH, HIST, EMBED_DIM), jnp.float32),
        mesh=plsc.VectorSubcoreMesh(core_axis_name="c", subcore_axis_name="s"),
        compiler_params=pltpu.CompilerParams(
            needs_layout_passes=False, use_tc_tiling_on_sc=False
        ),
        scratch_types=[
            pltpu.VMEM((ROWS_W, HIST), jnp.int32),
            pltpu.VMEM((NBUF, CR, HIST, EMBED_DIM), jnp.float32),
            pltpu.SemaphoreType.DMA,
            pltpu.SemaphoreType.DMA,
            pltpu.SemaphoreType.DMA,
            pltpu.SemaphoreType.DMA,
            pltpu.SemaphoreType.DMA,
            pltpu.SemaphoreType.DMA,
            pltpu.SemaphoreType.DMA,
            pltpu.SemaphoreType.DMA,
        ],
    )(x, weight)
    return out


# speedup vs baseline: 1.3560x; 1.3560x over previous
import jax
import jax.numpy as jnp
from jax import lax
from jax.experimental import pallas as pl
from jax.experimental.pallas import tpu as pltpu
from jax.experimental.pallas import tpu_sc as plsc

VOCAB = 1000000
EMBED_DIM = 32
BATCH = 16384
HIST = 50

NC, NS = 2, 16
NW = NC * NS
ROWS_W = BATCH // NW
CR = 8
N_CHUNKS = ROWS_W // CR
FLAT = CR * HIST
NBUF = 4
N_SUPER = N_CHUNKS // NBUF


def _rsqrt_newton(s):
    s = jnp.maximum(s, jnp.float32(1e-24))
    i = lax.bitcast_convert_type(s, jnp.int32)
    y = lax.bitcast_convert_type(jnp.int32(0x5F3759DF) - (i >> 1), jnp.float32)
    half_s = jnp.float32(0.5) * s
    for _ in range(2):
        y = y * (jnp.float32(1.5) - half_s * y * y)
    return y


def _sc_body(x_hbm, w_hbm, out_hbm, idxa, rows4, g0, g1, g2, g3, o0, o1, o2, o3):
    wid = lax.axis_index("s") * NC + lax.axis_index("c")
    gsem = (g0, g1, g2, g3)
    osem = (o0, o1, o2, o3)
    row0 = wid * ROWS_W

    pltpu.sync_copy(x_hbm.at[pl.ds(row0, ROWS_W), :], idxa)

    def fire_gather(c, b):
        for j in range(CR):
            pltpu.async_copy(
                w_hbm.at[idxa.at[c * CR + j]], rows4.at[b, j], gsem[b]
            )

    def wait_gather(b):
        for j in range(CR):
            pltpu.make_async_copy(
                w_hbm.at[pl.ds(0, HIST), :], rows4.at[b, j], gsem[b]
            ).wait()

    def out_copy(c, b):
        return pltpu.make_async_copy(
            rows4.at[b], out_hbm.at[pl.ds(row0 + c * CR, CR), :, :], osem[b]
        )

    perms = [lax.iota(jnp.int32, 16) ^ k for k in (8, 4, 2, 1)]
    _dnums = lax.GatherDimensionNumbers(
        offset_dims=(), collapsed_slice_dims=(0,), start_index_map=(0,)
    )

    def _permute(t, p):
        return lax.gather(
            t,
            p[:, None],
            dimension_numbers=_dnums,
            slice_sizes=(1,),
            mode=lax.GatherScatterMode.PROMISE_IN_BOUNDS,
        )

    def normalize(b):
        def one_row(i, h):
            a = rows4[b, i, h, pl.ds(0, 16)]
            bb = rows4[b, i, h, pl.ds(16, 16)]
            t = a * a + bb * bb
            for p in perms:
                t = t + _permute(t, p)
            y = _rsqrt_newton(t)
            rows4[b, i, h, pl.ds(0, 16)] = a * y
            rows4[b, i, h, pl.ds(16, 16)] = bb * y

        def i_body(i, carry):
            def h_body(hq, carry2):
                for k in range(5):
                    one_row(i, hq * 5 + k)
                return carry2

            lax.fori_loop(0, HIST // 5, h_body, 0)
            return carry

        lax.fori_loop(0, CR, i_body, 0)

    fire_gather(0, 0)
    fire_gather(1, 1)

    def super_body(s, carry):
        for i in range(NBUF):
            c = s * NBUF + i
            wait_gather(i)
            normalize(i)
            out_copy(c, i).start()
            bn = (i + 2) % NBUF

            @pl.when(c + 2 < N_CHUNKS)
            def _():
                @pl.when(c >= 2)
                def _():
                    out_copy(c - 2, bn).wait()

                fire_gather(c + 2, bn)

        return carry

    lax.fori_loop(0, N_SUPER, super_body, 0)
    for c in range(N_CHUNKS - NBUF, N_CHUNKS):
        out_copy(c, c % NBUF).wait()


@jax.jit
def kernel(x, weight):
    out = pl.kernel(
        _sc_body,
        out_type=jax.ShapeDtypeStruct((BATCH, HIST, EMBED_DIM), jnp.float32),
        mesh=plsc.VectorSubcoreMesh(core_axis_name="c", subcore_axis_name="s"),
        compiler_params=pltpu.CompilerParams(
            needs_layout_passes=False, use_tc_tiling_on_sc=False
        ),
        scratch_types=[
            pltpu.VMEM((ROWS_W, HIST), jnp.int32),
            pltpu.VMEM((NBUF, CR, HIST, EMBED_DIM), jnp.float32),
            pltpu.SemaphoreType.DMA,
            pltpu.SemaphoreType.DMA,
            pltpu.SemaphoreType.DMA,
            pltpu.SemaphoreType.DMA,
            pltpu.SemaphoreType.DMA,
            pltpu.SemaphoreType.DMA,
            pltpu.SemaphoreType.DMA,
            pltpu.SemaphoreType.DMA,
        ],
    )(x, weight)
    return out


# speedup vs baseline: 1.9633x; 1.4479x over previous
import jax
import jax.numpy as jnp
from jax import lax
from jax.experimental import pallas as pl
from jax.experimental.pallas import tpu as pltpu
from jax.experimental.pallas import tpu_sc as plsc

VOCAB = 1000000
EMBED_DIM = 32
BATCH = 16384
HIST = 50

NC, NS = 2, 16
NW = NC * NS
ROWS_W = BATCH // NW
CR = 8
N_CHUNKS = ROWS_W // CR
NBUF = 4
N_SUPER = N_CHUNKS // NBUF

LINES = BATCH * HIST * EMBED_DIM // 128
TC_BLOCK = 2048


def _sc_gather(x_hbm, w_hbm, out_hbm, idxa, rows4, g0, g1, g2, g3, o0, o1, o2, o3):
    wid = lax.axis_index("s") * NC + lax.axis_index("c")
    gsem = (g0, g1, g2, g3)
    osem = (o0, o1, o2, o3)
    row0 = wid * ROWS_W

    pltpu.sync_copy(x_hbm.at[pl.ds(row0, ROWS_W), :], idxa)

    def fire_gather(c, b):
        for j in range(CR):
            pltpu.async_copy(
                w_hbm.at[idxa.at[c * CR + j]], rows4.at[b, j], gsem[b]
            )

    def wait_gather(b):
        for j in range(CR):
            pltpu.make_async_copy(
                w_hbm.at[pl.ds(0, HIST), :], rows4.at[b, j], gsem[b]
            ).wait()

    def out_copy(c, b):
        return pltpu.make_async_copy(
            rows4.at[b], out_hbm.at[pl.ds(row0 + c * CR, CR), :, :], osem[b]
        )

    fire_gather(0, 0)
    fire_gather(1, 1)

    def super_body(s, carry):
        for i in range(NBUF):
            c = s * NBUF + i
            wait_gather(i)
            out_copy(c, i).start()
            bn = (i + 2) % NBUF

            @pl.when(c + 2 < N_CHUNKS)
            def _():
                @pl.when(c >= 2)
                def _():
                    out_copy(c - 2, bn).wait()

                fire_gather(c + 2, bn)

        return carry

    lax.fori_loop(0, N_SUPER, super_body, 0)
    for c in range(N_CHUNKS - NBUF, N_CHUNKS):
        out_copy(c, c % NBUF).wait()


def _tc_normalize(x_ref, o_ref):
    x = x_ref[...]
    r = lax.broadcasted_iota(jnp.int32, (128, 128), 0) // EMBED_DIM
    c = lax.broadcasted_iota(jnp.int32, (128, 128), 1) // EMBED_DIM
    seg = (r == c).astype(jnp.float32)
    s = jax.lax.dot(
        x * x,
        seg,
        precision=lax.Precision.HIGHEST,
        preferred_element_type=jnp.float32,
    )
    o_ref[...] = x * lax.rsqrt(jnp.maximum(s, jnp.float32(1e-24)))


@jax.jit
def kernel(x, weight):
    gathered = pl.kernel(
        _sc_gather,
        out_type=jax.ShapeDtypeStruct((BATCH, HIST, EMBED_DIM), jnp.float32),
        mesh=plsc.VectorSubcoreMesh(core_axis_name="c", subcore_axis_name="s"),
        compiler_params=pltpu.CompilerParams(
            needs_layout_passes=False, use_tc_tiling_on_sc=False
        ),
        scratch_types=[
            pltpu.VMEM((ROWS_W, HIST), jnp.int32),
            pltpu.VMEM((NBUF, CR, HIST, EMBED_DIM), jnp.float32),
            pltpu.SemaphoreType.DMA,
            pltpu.SemaphoreType.DMA,
            pltpu.SemaphoreType.DMA,
            pltpu.SemaphoreType.DMA,
            pltpu.SemaphoreType.DMA,
            pltpu.SemaphoreType.DMA,
            pltpu.SemaphoreType.DMA,
            pltpu.SemaphoreType.DMA,
        ],
    )(x, weight)

    lines = gathered.reshape(LINES, 128)
    normalized = pl.pallas_call(
        _tc_normalize,
        grid=(LINES // TC_BLOCK,),
        in_specs=[pl.BlockSpec((TC_BLOCK, 128), lambda i: (i, 0))],
        out_specs=pl.BlockSpec((TC_BLOCK, 128), lambda i: (i, 0)),
        out_shape=jax.ShapeDtypeStruct((LINES, 128), jnp.float32),
    )(lines)
    return normalized.reshape(BATCH, HIST, EMBED_DIM)


# speedup vs baseline: 2.0070x; 1.0222x over previous
import jax
import jax.numpy as jnp
from jax import lax
from jax.experimental import pallas as pl
from jax.experimental.pallas import tpu as pltpu
from jax.experimental.pallas import tpu_sc as plsc

VOCAB = 1000000
EMBED_DIM = 32
BATCH = 16384
HIST = 50

NC, NS = 2, 16
NW = NC * NS
ROWS_W = BATCH // NW
CR = 8
N_CHUNKS = ROWS_W // CR
NBUF = 4
N_SUPER = N_CHUNKS // NBUF

LINES = BATCH * HIST * EMBED_DIM // 128
TC_BLOCK = 2048


def _sc_gather(x_hbm, w_hbm, out_hbm, idxa, rows4, g0, g1, g2, g3, o0, o1, o2, o3):
    wid = lax.axis_index("s") * NC + lax.axis_index("c")
    gsem = (g0, g1, g2, g3)
    osem = (o0, o1, o2, o3)
    row0 = wid * ROWS_W

    pltpu.sync_copy(x_hbm.at[pl.ds(row0, ROWS_W), :], idxa)

    def fire_gather(c, b):
        for j in range(CR):
            pltpu.async_copy(
                w_hbm.at[idxa.at[c * CR + j]], rows4.at[b, j], gsem[b]
            )

    def wait_gather(b):
        for j in range(CR):
            pltpu.make_async_copy(
                w_hbm.at[pl.ds(0, HIST), :], rows4.at[b, j], gsem[b]
            ).wait()

    def out_copy(c, b):
        return pltpu.make_async_copy(
            rows4.at[b], out_hbm.at[pl.ds(row0 + c * CR, CR), :, :], osem[b]
        )

    fire_gather(0, 0)
    fire_gather(1, 1)

    def super_body(s, carry):
        for i in range(NBUF):
            c = s * NBUF + i
            wait_gather(i)
            out_copy(c, i).start()
            bn = (i + 2) % NBUF

            @pl.when(c + 2 < N_CHUNKS)
            def _():
                @pl.when(c >= 2)
                def _():
                    out_copy(c - 2, bn).wait()

                fire_gather(c + 2, bn)

        return carry

    lax.fori_loop(0, N_SUPER, super_body, 0)
    for c in range(N_CHUNKS - NBUF, N_CHUNKS):
        out_copy(c, c % NBUF).wait()


def _tc_normalize(x_ref, o_ref):
    x = x_ref[...]
    r = lax.broadcasted_iota(jnp.int32, (128, 128), 0) // EMBED_DIM
    c = lax.broadcasted_iota(jnp.int32, (128, 128), 1) // EMBED_DIM
    seg = (r == c).astype(jnp.float32)
    s = jax.lax.dot(
        x * x,
        seg,
        preferred_element_type=jnp.float32,
    )
    o_ref[...] = x * lax.rsqrt(jnp.maximum(s, jnp.float32(1e-24)))


@jax.jit
def kernel(x, weight):
    gathered = pl.kernel(
        _sc_gather,
        out_type=jax.ShapeDtypeStruct((BATCH, HIST, EMBED_DIM), jnp.float32),
        mesh=plsc.VectorSubcoreMesh(core_axis_name="c", subcore_axis_name="s"),
        compiler_params=pltpu.CompilerParams(
            needs_layout_passes=False, use_tc_tiling_on_sc=False
        ),
        scratch_types=[
            pltpu.VMEM((ROWS_W, HIST), jnp.int32),
            pltpu.VMEM((NBUF, CR, HIST, EMBED_DIM), jnp.float32),
            pltpu.SemaphoreType.DMA,
            pltpu.SemaphoreType.DMA,
            pltpu.SemaphoreType.DMA,
            pltpu.SemaphoreType.DMA,
            pltpu.SemaphoreType.DMA,
            pltpu.SemaphoreType.DMA,
            pltpu.SemaphoreType.DMA,
            pltpu.SemaphoreType.DMA,
        ],
    )(x, weight)

    lines = gathered.reshape(LINES, 128)
    normalized = pl.pallas_call(
        _tc_normalize,
        grid=(LINES // TC_BLOCK,),
        in_specs=[pl.BlockSpec((TC_BLOCK, 128), lambda i: (i, 0))],
        out_specs=pl.BlockSpec((TC_BLOCK, 128), lambda i: (i, 0)),
        out_shape=jax.ShapeDtypeStruct((LINES, 128), jnp.float32),
    )(lines)
    return normalized.reshape(BATCH, HIST, EMBED_DIM)


# speedup vs baseline: 2.0725x; 1.0327x over previous
import jax
import jax.numpy as jnp
from jax import lax
from jax.experimental import pallas as pl
from jax.experimental.pallas import tpu as pltpu
from jax.experimental.pallas import tpu_sc as plsc

VOCAB = 1000000
EMBED_DIM = 32
BATCH = 16384
HIST = 50

NC, NS = 2, 16
NW = NC * NS
ROWS_W = BATCH // NW
CR = 8
N_CHUNKS = ROWS_W // CR
NBUF = 4
N_SUPER = N_CHUNKS // NBUF

LINES = BATCH * HIST * EMBED_DIM // 128
TC_BLOCK = 4096


def _sc_gather(x_hbm, w_hbm, out_hbm, idxa, rows4, g0, g1, g2, g3, o0, o1, o2, o3):
    wid = lax.axis_index("s") * NC + lax.axis_index("c")
    gsem = (g0, g1, g2, g3)
    osem = (o0, o1, o2, o3)
    row0 = wid * ROWS_W

    pltpu.sync_copy(x_hbm.at[pl.ds(row0, ROWS_W), :], idxa)

    def fire_gather(c, b):
        for j in range(CR):
            pltpu.async_copy(
                w_hbm.at[idxa.at[c * CR + j]], rows4.at[b, j], gsem[b]
            )

    def wait_gather(b):
        for j in range(CR):
            pltpu.make_async_copy(
                w_hbm.at[pl.ds(0, HIST), :], rows4.at[b, j], gsem[b]
            ).wait()

    def out_copy(c, b):
        return pltpu.make_async_copy(
            rows4.at[b], out_hbm.at[pl.ds(row0 + c * CR, CR), :, :], osem[b]
        )

    fire_gather(0, 0)
    fire_gather(1, 1)

    def super_body(s, carry):
        for i in range(NBUF):
            c = s * NBUF + i
            wait_gather(i)
            out_copy(c, i).start()
            bn = (i + 2) % NBUF

            @pl.when(c + 2 < N_CHUNKS)
            def _():
                @pl.when(c >= 2)
                def _():
                    out_copy(c - 2, bn).wait()

                fire_gather(c + 2, bn)

        return carry

    lax.fori_loop(0, N_SUPER, super_body, 0)
    for c in range(N_CHUNKS - NBUF, N_CHUNKS):
        out_copy(c, c % NBUF).wait()


def _tc_normalize(x_ref, o_ref):
    x = x_ref[...]
    r = lax.broadcasted_iota(jnp.int32, (128, 128), 0) // EMBED_DIM
    c = lax.broadcasted_iota(jnp.int32, (128, 128), 1) // EMBED_DIM
    seg = (r == c).astype(jnp.float32)
    s = jax.lax.dot(
        x * x,
        seg,
        preferred_element_type=jnp.float32,
    )
    o_ref[...] = x * lax.rsqrt(jnp.maximum(s, jnp.float32(1e-24)))


@jax.jit
def kernel(x, weight):
    gathered = pl.kernel(
        _sc_gather,
        out_type=jax.ShapeDtypeStruct((BATCH, HIST, EMBED_DIM), jnp.float32),
        mesh=plsc.VectorSubcoreMesh(core_axis_name="c", subcore_axis_name="s"),
        compiler_params=pltpu.CompilerParams(
            needs_layout_passes=False, use_tc_tiling_on_sc=False
        ),
        scratch_types=[
            pltpu.VMEM((ROWS_W, HIST), jnp.int32),
            pltpu.VMEM((NBUF, CR, HIST, EMBED_DIM), jnp.float32),
            pltpu.SemaphoreType.DMA,
            pltpu.SemaphoreType.DMA,
            pltpu.SemaphoreType.DMA,
            pltpu.SemaphoreType.DMA,
            pltpu.SemaphoreType.DMA,
            pltpu.SemaphoreType.DMA,
            pltpu.SemaphoreType.DMA,
            pltpu.SemaphoreType.DMA,
        ],
    )(x, weight)

    lines = gathered.reshape(LINES, 128)
    normalized = pl.pallas_call(
        _tc_normalize,
        grid=(LINES // TC_BLOCK,),
        in_specs=[pl.BlockSpec((TC_BLOCK, 128), lambda i: (i, 0))],
        out_specs=pl.BlockSpec((TC_BLOCK, 128), lambda i: (i, 0)),
        out_shape=jax.ShapeDtypeStruct((LINES, 128), jnp.float32),
    )(lines)
    return normalized.reshape(BATCH, HIST, EMBED_DIM)


# speedup vs baseline: 2.0916x; 1.0092x over previous
import jax
import jax.numpy as jnp
from jax import lax
from jax.experimental import pallas as pl
from jax.experimental.pallas import tpu as pltpu
from jax.experimental.pallas import tpu_sc as plsc

VOCAB = 1000000
EMBED_DIM = 32
BATCH = 16384
HIST = 50

NC, NS = 2, 16
NW = NC * NS
ROWS_W = BATCH // NW
CR = 8
N_CHUNKS = ROWS_W // CR
NBUF = 4
N_SUPER = N_CHUNKS // NBUF

LINES = BATCH * HIST * EMBED_DIM // 128
TC_BLOCK = 8192


def _sc_gather(x_hbm, w_hbm, out_hbm, idxa, rows4, g0, g1, g2, g3, o0, o1, o2, o3):
    wid = lax.axis_index("s") * NC + lax.axis_index("c")
    gsem = (g0, g1, g2, g3)
    osem = (o0, o1, o2, o3)
    row0 = wid * ROWS_W

    pltpu.sync_copy(x_hbm.at[pl.ds(row0, ROWS_W), :], idxa)

    def fire_gather(c, b):
        for j in range(CR):
            pltpu.async_copy(
                w_hbm.at[idxa.at[c * CR + j]], rows4.at[b, j], gsem[b]
            )

    def wait_gather(b):
        for j in range(CR):
            pltpu.make_async_copy(
                w_hbm.at[pl.ds(0, HIST), :], rows4.at[b, j], gsem[b]
            ).wait()

    def out_copy(c, b):
        return pltpu.make_async_copy(
            rows4.at[b], out_hbm.at[pl.ds(row0 + c * CR, CR), :, :], osem[b]
        )

    fire_gather(0, 0)
    fire_gather(1, 1)

    def super_body(s, carry):
        for i in range(NBUF):
            c = s * NBUF + i
            wait_gather(i)
            out_copy(c, i).start()
            bn = (i + 2) % NBUF

            @pl.when(c + 2 < N_CHUNKS)
            def _():
                @pl.when(c >= 2)
                def _():
                    out_copy(c - 2, bn).wait()

                fire_gather(c + 2, bn)

        return carry

    lax.fori_loop(0, N_SUPER, super_body, 0)
    for c in range(N_CHUNKS - NBUF, N_CHUNKS):
        out_copy(c, c % NBUF).wait()


def _tc_normalize(x_ref, o_ref):
    x = x_ref[...]
    r = lax.broadcasted_iota(jnp.int32, (128, 128), 0) // EMBED_DIM
    c = lax.broadcasted_iota(jnp.int32, (128, 128), 1) // EMBED_DIM
    seg = (r == c).astype(jnp.float32)
    s = jax.lax.dot(
        x * x,
        seg,
        preferred_element_type=jnp.float32,
    )
    o_ref[...] = x * lax.rsqrt(jnp.maximum(s, jnp.float32(1e-24)))


@jax.jit
def kernel(x, weight):
    gathered = pl.kernel(
        _sc_gather,
        out_type=jax.ShapeDtypeStruct((BATCH, HIST, EMBED_DIM), jnp.float32),
        mesh=plsc.VectorSubcoreMesh(core_axis_name="c", subcore_axis_name="s"),
        compiler_params=pltpu.CompilerParams(
            needs_layout_passes=False, use_tc_tiling_on_sc=False
        ),
        scratch_types=[
            pltpu.VMEM((ROWS_W, HIST), jnp.int32),
            pltpu.VMEM((NBUF, CR, HIST, EMBED_DIM), jnp.float32),
            pltpu.SemaphoreType.DMA,
            pltpu.SemaphoreType.DMA,
            pltpu.SemaphoreType.DMA,
            pltpu.SemaphoreType.DMA,
            pltpu.SemaphoreType.DMA,
            pltpu.SemaphoreType.DMA,
            pltpu.SemaphoreType.DMA,
            pltpu.SemaphoreType.DMA,
        ],
    )(x, weight)

    lines = gathered.reshape(LINES, 128)
    normalized = pl.pallas_call(
        _tc_normalize,
        grid=(LINES // TC_BLOCK,),
        in_specs=[pl.BlockSpec((TC_BLOCK, 128), lambda i: (i, 0))],
        out_specs=pl.BlockSpec((TC_BLOCK, 128), lambda i: (i, 0)),
        out_shape=jax.ShapeDtypeStruct((LINES, 128), jnp.float32),
    )(lines)
    return normalized.reshape(BATCH, HIST, EMBED_DIM)
